# baseline (speedup 1.0000x reference)
import functools

import jax
import jax.numpy as jnp
from jax import lax
from jax.experimental import pallas as pl
from jax.experimental.pallas import tpu as pltpu
from jax.experimental.pallas import tpu_sc as plsc

HIDDEN = 4096
NUM_EXPERTS = 64
TOP_K = 8
TOKEN_TILE = 1024

_NC = 2
_NS = 16
_NW = _NC * _NS
_SLAB = 128


def _logits_kernel(x_ref, w_ref, out_ref):
    out_ref[...] = jax.lax.dot_general(
        x_ref[...], w_ref[...], (((1,), (1,)), ((), ())),
        preferred_element_type=jnp.float32,
    )


_GATHER_DNUMS = lax.GatherDimensionNumbers(
    offset_dims=(), collapsed_slice_dims=(0,), start_index_map=(0,))


def _dyn_gather(x, idx):
    return lax.gather(x, idx[:, None], _GATHER_DNUMS, (1,),
                      mode=lax.GatherScatterMode.PROMISE_IN_BOUNDS)


def _vmax_splat(x):
    iota = lax.iota(jnp.int32, 16)
    for s in (8, 4, 2, 1):
        x = jnp.maximum(x, _dyn_gather(x, jnp.bitwise_xor(iota, s)))
    return x


def _route_one_token(in_v, pr_v, mp_v, off):
    neg_inf = jnp.float32(-jnp.inf)
    vs = [in_v[pl.ds(off + k * 16, 16)] for k in range(4)]
    avail = list(vs)
    m0 = None
    denom = None
    for _ in range(TOP_K):
        m = jnp.maximum(jnp.maximum(avail[0], avail[1]),
                        jnp.maximum(avail[2], avail[3]))
        mxv = _vmax_splat(m)
        avail = [jnp.where(v == mxv, neg_inf, v) for v in avail]
        if m0 is None:
            m0 = mxv
            denom = jnp.ones((16,), dtype=jnp.float32)
        else:
            denom = denom + jnp.exp(mxv - m0)
    scale = 1.0 / denom
    for k in range(4):
        e = jnp.exp(vs[k] - m0)
        pr_v[pl.ds(off + k * 16, 16)] = e * scale
        mp_v[pl.ds(off + k * 16, 16)] = jnp.where(
            avail[k] == neg_inf, jnp.float32(1.0), jnp.float32(0.0))


def _make_sc_router(n_tokens):
    tok_per_w = n_tokens // _NW
    n_slabs = tok_per_w // _SLAB
    mesh = plsc.VectorSubcoreMesh(core_axis_name="c", subcore_axis_name="s")

    @functools.partial(
        pl.kernel, mesh=mesh,
        out_type=[
            jax.ShapeDtypeStruct((n_tokens * NUM_EXPERTS,), jnp.float32),
            jax.ShapeDtypeStruct((n_tokens * NUM_EXPERTS,), jnp.float32),
        ],
        scratch_types=[
            pltpu.VMEM((_SLAB * NUM_EXPERTS,), jnp.float32),
            pltpu.VMEM((_SLAB * NUM_EXPERTS,), jnp.float32),
            pltpu.VMEM((_SLAB * NUM_EXPERTS,), jnp.float32),
        ],
    )
    def _route(logits_hbm, probs_hbm, map_hbm, in_v, pr_v, mp_v):
        wid = lax.axis_index("s") * _NC + lax.axis_index("c")
        base_w = pl.multiple_of(wid * (tok_per_w * NUM_EXPERTS),
                                _SLAB * NUM_EXPERTS)

        def slab_body(j, carry):
            base = pl.multiple_of(base_w + j * (_SLAB * NUM_EXPERTS),
                                  _SLAB * NUM_EXPERTS)
            pltpu.sync_copy(logits_hbm.at[pl.ds(base, _SLAB * NUM_EXPERTS)],
                            in_v)

            def tok_body(t, c):
                off = pl.multiple_of(t * (2 * NUM_EXPERTS), 2 * NUM_EXPERTS)
                _route_one_token(in_v, pr_v, mp_v, off)
                _route_one_token(in_v, pr_v, mp_v, off + NUM_EXPERTS)
                return c

            lax.fori_loop(0, _SLAB // 2, tok_body, 0)
            pltpu.sync_copy(pr_v,
                            probs_hbm.at[pl.ds(base, _SLAB * NUM_EXPERTS)])
            pltpu.sync_copy(mp_v,
                            map_hbm.at[pl.ds(base, _SLAB * NUM_EXPERTS)])
            return carry

        lax.fori_loop(0, n_slabs, slab_body, 0)

    return _route


def kernel(x, W_gate):
    n_tokens = x.shape[0]
    grid = (n_tokens // TOKEN_TILE,)
    logits = pl.pallas_call(
        _logits_kernel,
        grid=grid,
        in_specs=[
            pl.BlockSpec((TOKEN_TILE, HIDDEN), lambda i: (i, 0)),
            pl.BlockSpec((NUM_EXPERTS, HIDDEN), lambda i: (0, 0)),
        ],
        out_specs=pl.BlockSpec((TOKEN_TILE, NUM_EXPERTS), lambda i: (i, 0)),
        out_shape=jax.ShapeDtypeStruct((n_tokens, NUM_EXPERTS), jnp.float32),
        compiler_params=pltpu.CompilerParams(
            dimension_semantics=("parallel",),
        ),
    )(x, W_gate)
    probs_flat, map_flat = _make_sc_router(n_tokens)(logits.reshape(-1))
    probs = probs_flat.reshape(n_tokens, NUM_EXPERTS)
    routing_map = map_flat.reshape(n_tokens, NUM_EXPERTS) != 0.0
    return probs, routing_map


# speedup vs baseline: 1.4653x; 1.4653x over previous
import functools

import jax
import jax.numpy as jnp
from jax.experimental import pallas as pl
from jax.experimental.pallas import tpu as pltpu

HIDDEN = 4096
NUM_EXPERTS = 64
TOP_K = 8
TOKEN_TILE = 1024


def _router_kernel(x_ref, w_ref, probs_ref, map_ref):
    x = x_ref[...]
    w = w_ref[...]
    logits = jax.lax.dot_general(
        x, w, (((1,), (1,)), ((), ())),
        preferred_element_type=jnp.float32,
    )
    n = logits.shape[0]
    neg_inf = jnp.float32(-jnp.inf)

    selected = jnp.zeros((n, NUM_EXPERTS), dtype=jnp.bool_)
    rowmax = None
    denom = None
    for _ in range(TOP_K):
        avail = jnp.where(selected, neg_inf, logits)
        m = jnp.max(avail, axis=1, keepdims=True)
        selected = jnp.logical_or(selected, avail == m)
        if rowmax is None:
            rowmax = m
            denom = jnp.ones_like(m)
        else:
            denom = denom + jnp.exp(m - rowmax)

    e = jnp.exp(logits - rowmax)
    probs_ref[...] = e * (1.0 / denom)
    map_ref[...] = selected


@functools.partial(jax.jit, static_argnames=())
def kernel(x, W_gate):
    n_tokens = x.shape[0]
    grid = (n_tokens // TOKEN_TILE,)
    probs, map_f32 = pl.pallas_call(
        _router_kernel,
        grid=grid,
        in_specs=[
            pl.BlockSpec((TOKEN_TILE, HIDDEN), lambda i: (i, 0)),
            pl.BlockSpec((NUM_EXPERTS, HIDDEN), lambda i: (0, 0)),
        ],
        out_specs=[
            pl.BlockSpec((TOKEN_TILE, NUM_EXPERTS), lambda i: (i, 0)),
            pl.BlockSpec((TOKEN_TILE, NUM_EXPERTS), lambda i: (i, 0)),
        ],
        out_shape=[
            jax.ShapeDtypeStruct((n_tokens, NUM_EXPERTS), jnp.float32),
            jax.ShapeDtypeStruct((n_tokens, NUM_EXPERTS), jnp.bool_),
        ],
        compiler_params=pltpu.CompilerParams(
            dimension_semantics=("parallel",),
        ),
    )(x, W_gate)
    return probs, map_f32


# speedup vs baseline: 1.4682x; 1.0020x over previous
import functools

import jax
import jax.numpy as jnp
from jax.experimental import pallas as pl
from jax.experimental.pallas import tpu as pltpu

HIDDEN = 4096
NUM_EXPERTS = 64
TOP_K = 8
TOKEN_TILE = 1024


def _router_kernel(xa_ref, xb_ref, w_ref, probs_ref, map_ref):
    w = w_ref[...]
    wa = w[:, : HIDDEN // 2]
    wb = w[:, HIDDEN // 2 :]
    dn = (((1,), (1,)), ((), ()))
    logits = (
        jax.lax.dot_general(xa_ref[...], wa, dn,
                            preferred_element_type=jnp.float32)
        + jax.lax.dot_general(xb_ref[...], wb, dn,
                              preferred_element_type=jnp.float32)
    )
    n = logits.shape[0]
    neg_inf = jnp.float32(-jnp.inf)

    selected = jnp.zeros((n, NUM_EXPERTS), dtype=jnp.bool_)
    rowmax = None
    denom = None
    for _ in range(TOP_K):
        avail = jnp.where(selected, neg_inf, logits)
        m = jnp.max(avail, axis=1, keepdims=True)
        selected = jnp.logical_or(selected, avail == m)
        if rowmax is None:
            rowmax = m
            denom = jnp.ones_like(m)
        else:
            denom = denom + jnp.exp(m - rowmax)

    e = jnp.exp(logits - rowmax)
    probs_ref[...] = e * (1.0 / denom)
    map_ref[...] = selected


@functools.partial(jax.jit, static_argnames=())
def kernel(x, W_gate):
    n_tokens = x.shape[0]
    grid = (n_tokens // TOKEN_TILE,)
    probs, map_f32 = pl.pallas_call(
        _router_kernel,
        grid=grid,
        in_specs=[
            pl.BlockSpec((TOKEN_TILE, HIDDEN // 2), lambda i: (i, 0)),
            pl.BlockSpec((TOKEN_TILE, HIDDEN // 2), lambda i: (i, 1)),
            pl.BlockSpec((NUM_EXPERTS, HIDDEN), lambda i: (0, 0)),
        ],
        out_specs=[
            pl.BlockSpec((TOKEN_TILE, NUM_EXPERTS), lambda i: (i, 0)),
            pl.BlockSpec((TOKEN_TILE, NUM_EXPERTS), lambda i: (i, 0)),
        ],
        out_shape=[
            jax.ShapeDtypeStruct((n_tokens, NUM_EXPERTS), jnp.float32),
            jax.ShapeDtypeStruct((n_tokens, NUM_EXPERTS), jnp.bool_),
        ],
        compiler_params=pltpu.CompilerParams(
            dimension_semantics=("parallel",),
        ),
    )(x, x, W_gate)
    return probs, map_f32
